# Initial kernel scaffold; baseline (speedup 1.0000x reference)
import jax
import jax.numpy as jnp
from jax.experimental import pallas as pl


def kernel(idx, table):
    raise NotImplementedError("write your pallas kernel here")


# speedup vs baseline: 1.0148x; 1.0148x over previous
import functools

import jax
import jax.numpy as jnp
from jax import lax
from jax.experimental import pallas as pl
from jax.experimental.pallas import tpu as pltpu
from jax.experimental.pallas import tpu_sc as plsc

VOCAB = 1000
D = 1000
NC = 2
NS = 16
NW = NC * NS
CHUNK = 64


@functools.partial(jax.jit, static_argnames=("b_total",))
def _gather_rows(idx_flat, table, b_total):
    b_per_w = b_total // NW
    n_chunks = b_per_w // CHUNK
    mesh = plsc.VectorSubcoreMesh(
        core_axis_name="c", subcore_axis_name="s", num_cores=NC, num_subcores=NS
    )

    @functools.partial(
        pl.kernel,
        mesh=mesh,
        out_type=jax.ShapeDtypeStruct((b_total, D), jnp.float32),
        scratch_types=[
            pltpu.VMEM((b_per_w,), jnp.int32),
            pltpu.VMEM((CHUNK, D), jnp.float32),
            pltpu.SemaphoreType.DMA,
        ],
        compiler_params=pltpu.CompilerParams(use_tc_tiling_on_sc=False),
    )
    def k(idx_hbm, table_hbm, out_hbm, idx_v, rows_v, sem):
        wid = lax.axis_index("s") * NC + lax.axis_index("c")
        base = wid * b_per_w
        pltpu.sync_copy(idx_hbm.at[pl.ds(base, b_per_w)], idx_v)

        def body(c, carry):
            cp = pltpu.async_copy(
                table_hbm.at[idx_v.at[pl.ds(c * CHUNK, CHUNK)]], rows_v, sem
            )
            cp.wait()
            pltpu.sync_copy(rows_v, out_hbm.at[pl.ds(base + c * CHUNK, CHUNK)])
            return carry

        lax.fori_loop(0, n_chunks, body, 0)

    return k(idx_flat, table)


def kernel(idx, table):
    b, t = idx.shape
    flat = _gather_rows(idx.reshape(b * t), table, b * t)
    return flat.reshape(b, t, D)


# speedup vs baseline: 1.0271x; 1.0121x over previous
import functools

import jax
import jax.numpy as jnp
from jax import lax
from jax.experimental import pallas as pl
from jax.experimental.pallas import tpu as pltpu
from jax.experimental.pallas import tpu_sc as plsc

VOCAB = 1000
D = 1000
NC = 2
NS = 16
NW = NC * NS
CHUNK = 16
NBUF = 4


@functools.partial(jax.jit, static_argnames=("b_total",))
def _gather_rows(idx_flat, table, b_total):
    b_per_w = b_total // NW
    n_chunks = b_per_w // CHUNK
    n_outer = (n_chunks - NBUF) // NBUF
    mesh = plsc.VectorSubcoreMesh(
        core_axis_name="c", subcore_axis_name="s", num_cores=NC, num_subcores=NS
    )

    @functools.partial(
        pl.kernel,
        mesh=mesh,
        out_type=jax.ShapeDtypeStruct((b_total, D), jnp.float32),
        scratch_types=[
            pltpu.VMEM((b_per_w,), jnp.int32),
            pltpu.VMEM((NBUF, CHUNK, D), jnp.float32),
            [pltpu.SemaphoreType.DMA] * NBUF,
            [pltpu.SemaphoreType.DMA] * NBUF,
        ],
        compiler_params=pltpu.CompilerParams(use_tc_tiling_on_sc=False),
    )
    def k(idx_hbm, table_hbm, out_hbm, idx_v, rows_v, gsems, wsems):
        wid = lax.axis_index("s") * NC + lax.axis_index("c")
        base = wid * b_per_w
        pltpu.sync_copy(idx_hbm.at[pl.ds(base, b_per_w)], idx_v)

        def gather_cp(c, s):
            return pltpu.make_async_copy(
                table_hbm.at[idx_v.at[pl.ds(c * CHUNK, CHUNK)]],
                rows_v.at[s],
                gsems[s],
            )

        def write_cp(c, s):
            return pltpu.make_async_copy(
                rows_v.at[s],
                out_hbm.at[pl.ds(base + c * CHUNK, CHUNK)],
                wsems[s],
            )

        for s in range(NBUF):
            gather_cp(s, s).start()

        def body(j, carry):
            c0 = j * NBUF
            for s in range(NBUF):
                gather_cp(c0 + s, s).wait()
                write_cp(c0 + s, s).start()
            for s in range(NBUF):
                write_cp(c0 + s, s).wait()
                gather_cp(c0 + NBUF + s, s).start()
            return carry

        lax.fori_loop(0, n_outer, body, 0)

        c0 = n_chunks - NBUF
        for s in range(NBUF):
            gather_cp(c0 + s, s).wait()
            write_cp(c0 + s, s).start()
        for s in range(NBUF):
            write_cp(c0 + s, s).wait()

    return k(idx_flat, table)


def kernel(idx, table):
    b, t = idx.shape
    flat = _gather_rows(idx.reshape(b * t), table, b * t)
    return flat.reshape(b, t, D)


# speedup vs baseline: 1.0348x; 1.0075x over previous
import functools

import jax
import jax.numpy as jnp
from jax import lax
from jax.experimental import pallas as pl
from jax.experimental.pallas import tpu as pltpu
from jax.experimental.pallas import tpu_sc as plsc

VOCAB = 1000
D = 1000
NC = 2
NS = 16
NW = NC * NS
CHUNK = 40


@functools.partial(jax.jit, static_argnames=("b_total",))
def _gather_rows(idx_flat, table, b_total):
    b_per_w = b_total // NW
    n_chunks = b_per_w // CHUNK
    n_pairs = n_chunks // 2
    mesh = plsc.VectorSubcoreMesh(
        core_axis_name="c", subcore_axis_name="s", num_cores=NC, num_subcores=NS
    )

    @functools.partial(
        pl.kernel,
        mesh=mesh,
        out_type=jax.ShapeDtypeStruct((b_total, D), jnp.float32),
        scratch_types=[
            pltpu.VMEM((b_per_w,), jnp.int32),
            pltpu.VMEM((2, CHUNK, D), jnp.float32),
            [pltpu.SemaphoreType.DMA] * 2,
        ],
        compiler_params=pltpu.CompilerParams(use_tc_tiling_on_sc=False),
    )
    def k(idx_hbm, table_hbm, out_hbm, idx_v, rows_v, gsems):
        wid = lax.axis_index("s") * NC + lax.axis_index("c")
        base = wid * b_per_w
        pltpu.sync_copy(idx_hbm.at[pl.ds(base, b_per_w)], idx_v)

        def gather_cp(c, s):
            return pltpu.make_async_copy(
                table_hbm.at[idx_v.at[pl.ds(c * CHUNK, CHUNK)]],
                rows_v.at[s],
                gsems[s],
            )

        def write_out(c, s):
            pltpu.sync_copy(rows_v.at[s], out_hbm.at[pl.ds(base + c * CHUNK, CHUNK)])

        gather_cp(0, 0).start()
        gather_cp(1, 1).start()

        def body(p, carry):
            c = 2 * p
            gather_cp(c, 0).wait()
            write_out(c, 0)
            gather_cp(c + 2, 0).start()
            gather_cp(c + 1, 1).wait()
            write_out(c + 1, 1)
            gather_cp(c + 3, 1).start()
            return carry

        lax.fori_loop(0, n_pairs - 1, body, 0)

        c = n_chunks - 2
        gather_cp(c, 0).wait()
        write_out(c, 0)
        gather_cp(c + 1, 1).wait()
        write_out(c + 1, 1)

    return k(idx_flat, table)


def kernel(idx, table):
    b, t = idx.shape
    flat = _gather_rows(idx.reshape(b * t), table, b * t)
    return flat.reshape(b, t, D)
